# baseline (speedup 1.0000x reference)
import functools

import jax
import jax.numpy as jnp
import numpy as np
from jax import lax
from jax.experimental import pallas as pl
from jax.experimental.pallas import tpu as pltpu
from jax.experimental.pallas import tpu_sc as plsc

_H = 128
_W = 128
_FOCAL = 150.0
_CX = 63.5
_CY = 63.5
_K = 20
_THR = 0.01
_N = 1024
_P = _H * _W
_S = 9216
_PSC = _P - _S
_NW = 32
_RW = _PSC // _NW
_NC = _N // 16
_CAP = _N + 16
_RB = 512
_F0 = np.float32(0.0)
_F1 = np.float32(1.0)


def _ray_dirs():
    ys, xs = jnp.meshgrid(jnp.arange(_H, dtype=jnp.float32),
                          jnp.arange(_W, dtype=jnp.float32), indexing='ij')
    d = jnp.stack([(xs - _CX) / _FOCAL, (ys - _CY) / _FOCAL, jnp.ones_like(xs)], axis=-1)
    d = d / jnp.linalg.norm(d, axis=-1, keepdims=True)
    return d.reshape(-1, 3)


def _prep_body(r_ref, rp_ref, r2p_ref, g_ref,
               ro_ref, rpo_ref, r2po_ref, go_ref):
    ro_ref[...] = r_ref[...].astype(jnp.float32)
    rpo_ref[...] = rp_ref[...].astype(jnp.float32)
    r2po_ref[...] = r2p_ref[...].astype(jnp.float32)
    go_ref[...] = g_ref[...].astype(jnp.float32)


def _ld1(ref, off):
    return ref[pl.ds(off, 16)][0]


def _sc_body(rays_hbm, gauss_hbm, amm_hbm,
             w_hbm, idx_hbm, vnum_hbm, t_hbm,
             rxv, ryv, rzv, rx2v, ry2v, rz2v,
             gxv, gyv, gzv, gav, ammv,
             rowA, rowT, keyb, selAb, selTb, selIb, sortAb, sortTb,
             wst, ist, tst, vnst):
    wid = lax.axis_index("s") * 2 + lax.axis_index("c")
    base = wid * _RW

    for j, ref in enumerate((rxv, ryv, rzv, rx2v, ry2v, rz2v)):
        pltpu.sync_copy(rays_hbm.at[pl.ds(j * _P + _S + base, _RW)],
                        ref.at[pl.ds(0, _RW)])
    for j, ref in enumerate((gxv, gyv, gzv, gav)):
        pltpu.sync_copy(gauss_hbm.at[pl.ds(j * _N, _N)], ref)
    pltpu.sync_copy(amm_hbm, ammv)

    iota = lax.iota(jnp.int32, 16)
    iota_hi = iota + 16
    zf = jnp.zeros((16,), jnp.float32)
    zi = jnp.zeros((16,), jnp.int32)
    one_i = jnp.full((16,), 1, jnp.int32)
    perms = [iota ^ sh for sh in (8, 4, 2, 1)]
    _pib = lax.GatherScatterMode.PROMISE_IN_BOUNDS

    def _shuf(v, perm):
        return v.at[perm].get(mode=_pib)

    def _bfmax(v):
        for perm in perms:
            v = jnp.maximum(v, _shuf(v, perm))
        return v

    def _bfmin(v):
        for perm in perms:
            v = jnp.minimum(v, _shuf(v, perm))
        return v

    def _bfsum(v):
        for perm in perms:
            v = v + _shuf(v, perm)
        return v

    big_i = jnp.full((16,), 1 << 30, jnp.int32)
    big16 = jnp.full((16,), 16, jnp.int32)

    def ray_body(p, vn):
        rx = rxv[pl.ds(p, 16)][0]
        ry = ryv[pl.ds(p, 16)][0]
        rz = rzv[pl.ds(p, 16)][0]
        rx2 = rx2v[pl.ds(p, 16)][0]
        ry2 = ry2v[pl.ds(p, 16)][0]
        rz2 = rz2v[pl.ds(p, 16)][0]

        def chunk_body(c, carry):
            m0, m1, m2, m3 = carry
            b = c * 16
            vx = gxv[pl.ds(b, 16)]
            vy = gyv[pl.ds(b, 16)]
            vz = gzv[pl.ds(b, 16)]
            va = gav[pl.ds(b, 16)]
            vm = ammv[pl.ds(b, 16)]
            muAr = (rx * vx + ry * vy) + rz * vz
            rAr = (rx2 * va + ry2 * va) + rz2 * va
            tv = muAr / rAr
            quad = vm - (muAr * muAr) / rAr
            actv = jnp.exp(-0.5 * quad)
            ok = (actv > _THR) & (tv > 0.0)
            am = jnp.where(ok, actv, _F0)
            rowA[pl.ds(b, 16)] = am
            rowT[pl.ds(b, 16)] = tv
            cm = _bfmax(am)
            m0 = jnp.where(iota == c, cm, m0)
            m1 = jnp.where(iota == c - 16, cm, m1)
            m2 = jnp.where(iota == c - 32, cm, m2)
            m3 = jnp.where(iota == c - 48, cm, m3)
            return (m0, m1, m2, m3)

        m0, m1, m2, m3 = lax.fori_loop(
            0, _NC, chunk_body, (zf, zf, zf, zf))

        def sel_body(k, carry):
            m0, m1, m2, m3, sA0, sA1, sT0, sT1, sI0, sI1 = carry
            mx = _bfmax(jnp.maximum(jnp.maximum(m0, m1), jnp.maximum(m2, m3)))
            c0 = jnp.where(m0 == mx, iota, big_i)
            c1 = jnp.where(m1 == mx, iota + 16, big_i)
            c2 = jnp.where(m2 == mx, iota + 32, big_i)
            c3 = jnp.where(m3 == mx, iota + 48, big_i)
            cs_vec = _bfmin(jnp.minimum(jnp.minimum(c0, c1), jnp.minimum(c2, c3)))
            cs = cs_vec[0]
            bb = cs * 16
            v = rowA[pl.ds(bb, 16)]
            ln_vec = _bfmin(jnp.where(v == mx, iota, big16))
            pos_vec = cs_vec * 16 + ln_vec
            tld = rowT[pl.ds(bb, 16)]
            tsel = _shuf(tld, ln_vec)
            v2 = jnp.where(iota == ln_vec, np.float32(-1.0), v)
            rowA[pl.ds(bb, 16)] = v2
            nm = _bfmax(v2)
            m0 = jnp.where(iota == cs, nm, m0)
            m1 = jnp.where(iota == cs - 16, nm, m1)
            m2 = jnp.where(iota == cs - 32, nm, m2)
            m3 = jnp.where(iota == cs - 48, nm, m3)
            sA0 = jnp.where(iota == k, mx, sA0)
            sA1 = jnp.where(iota_hi == k, mx, sA1)
            sT0 = jnp.where(iota == k, tsel, sT0)
            sT1 = jnp.where(iota_hi == k, tsel, sT1)
            sI0 = jnp.where(iota == k, pos_vec, sI0)
            sI1 = jnp.where(iota_hi == k, pos_vec, sI1)
            return (m0, m1, m2, m3, sA0, sA1, sT0, sT1, sI0, sI1)

        (_, _, _, _, sA0, sA1, sT0, sT1, sI0, sI1) = lax.fori_loop(
            0, _K, sel_body, (m0, m1, m2, m3, zf, zf, zf, zf, zi, zi))

        nvalid = _bfsum(jnp.where(sA0 > 0.0, one_i, zi)
                        + jnp.where((sA1 > 0.0) & (iota_hi < _K), one_i, zi))

        inf = np.float32(np.inf)
        key0 = jnp.where(sA0 > 0.0, sT0, inf)
        key1 = jnp.where((sA1 > 0.0) & (iota_hi < _K), sT1, inf)
        keyb[pl.ds(0, 16)] = key0
        keyb[pl.ds(16, 16)] = key1
        selAb[pl.ds(0, 16)] = sA0
        selAb[pl.ds(16, 16)] = sA1
        selTb[pl.ds(0, 16)] = sT0
        selTb[pl.ds(16, 16)] = sT1
        selIb[pl.ds(0, 16)] = sI0
        selIb[pl.ds(16, 16)] = sI1

        def rank_body(i, carry):
            oA0, oA1, oT0, oT1, oI0, oI1 = carry
            ki = _ld1(keyb, i)
            cnt = (jnp.where(key0 < ki, one_i, zi)
                   + jnp.where(key1 < ki, one_i, zi)
                   + jnp.where((key0 == ki) & (iota < i), one_i, zi)
                   + jnp.where((key1 == ki) & (iota_hi < i), one_i, zi))
            rank_vec = _bfsum(cnt)
            ai = _ld1(selAb, i)
            ti = _ld1(selTb, i)
            ii = _ld1(selIb, i)
            oA0 = jnp.where(iota == rank_vec, ai, oA0)
            oA1 = jnp.where(iota_hi == rank_vec, ai, oA1)
            oT0 = jnp.where(iota == rank_vec, ti, oT0)
            oT1 = jnp.where(iota_hi == rank_vec, ti, oT1)
            oI0 = jnp.where(iota == rank_vec, ii, oI0)
            oI1 = jnp.where(iota_hi == rank_vec, ii, oI1)
            return (oA0, oA1, oT0, oT1, oI0, oI1)

        oA0, oA1, oT0, oT1, oI0, oI1 = lax.fori_loop(
            0, _K, rank_body, (zf, zf, zf, zf, zi, zi))
        sortAb[pl.ds(0, 16)] = oA0
        sortAb[pl.ds(16, 16)] = oA1
        sortTb[pl.ds(0, 16)] = oT0
        sortTb[pl.ds(16, 16)] = oT1

        def comp_body(k2, carry):
            trans, w0, w1, t0, t1 = carry
            av = _ld1(sortAb, k2)
            tvs = _ld1(sortTb, k2)
            alpha = jnp.minimum(jnp.maximum(av, _F0), np.float32(0.9999))
            wv = alpha * trans
            tv2 = jnp.where(av > 0.0, tvs, _F0)
            w0 = jnp.where(iota == k2, wv, w0)
            w1 = jnp.where(iota_hi == k2, wv, w1)
            t0 = jnp.where(iota == k2, tv2, t0)
            t1 = jnp.where(iota_hi == k2, tv2, t1)
            return (trans * (1.0 - alpha), w0, w1, t0, t1)

        _, w0, w1, t0, t1 = lax.fori_loop(
            0, _K, comp_body, (_F1, zf, zf, zf, zf))

        obase = p * _K
        wst[pl.ds(obase, 16)] = w0
        wst[pl.ds(obase + 16, 16)] = w1
        ist[pl.ds(obase, 16)] = oI0
        ist[pl.ds(obase + 16, 16)] = oI1
        tst[pl.ds(obase, 16)] = t0
        tst[pl.ds(obase + 16, 16)] = t1

        vn = jnp.where(iota == (p & 15), nvalid, vn)
        vnst[pl.ds((p >> 4) * 16, 16)] = vn
        return vn

    lax.fori_loop(0, _RW, ray_body, zi)

    pltpu.sync_copy(wst.at[pl.ds(0, _RW * _K)], w_hbm.at[pl.ds(base * _K, _RW * _K)])
    pltpu.sync_copy(ist.at[pl.ds(0, _RW * _K)], idx_hbm.at[pl.ds(base * _K, _RW * _K)])
    pltpu.sync_copy(tst.at[pl.ds(0, _RW * _K)], t_hbm.at[pl.ds(base * _K, _RW * _K)])
    pltpu.sync_copy(vnst.at[pl.ds(0, _RW)], vnum_hbm.at[pl.ds(base, _RW)])


_sc_render = functools.partial(
    pl.kernel,
    mesh=plsc.VectorSubcoreMesh(core_axis_name="c", subcore_axis_name="s"),
    out_type=[
        jax.ShapeDtypeStruct((_PSC * _K,), jnp.float32),
        jax.ShapeDtypeStruct((_PSC * _K,), jnp.int32),
        jax.ShapeDtypeStruct((_PSC,), jnp.int32),
        jax.ShapeDtypeStruct((_PSC * _K,), jnp.float32),
    ],
    scratch_types=[
        pltpu.VMEM((_RW + 16,), jnp.float32),
        pltpu.VMEM((_RW + 16,), jnp.float32),
        pltpu.VMEM((_RW + 16,), jnp.float32),
        pltpu.VMEM((_RW + 16,), jnp.float32),
        pltpu.VMEM((_RW + 16,), jnp.float32),
        pltpu.VMEM((_RW + 16,), jnp.float32),
        pltpu.VMEM((_N,), jnp.float32),
        pltpu.VMEM((_N,), jnp.float32),
        pltpu.VMEM((_N,), jnp.float32),
        pltpu.VMEM((_N,), jnp.float32),
        pltpu.VMEM((_N,), jnp.float32),
        pltpu.VMEM((_CAP,), jnp.float32),
        pltpu.VMEM((_CAP,), jnp.float32),
        pltpu.VMEM((48,), jnp.float32),
        pltpu.VMEM((48,), jnp.float32),
        pltpu.VMEM((48,), jnp.float32),
        pltpu.VMEM((48,), jnp.int32),
        pltpu.VMEM((48,), jnp.float32),
        pltpu.VMEM((48,), jnp.float32),
        pltpu.VMEM((_RW * _K + 16,), jnp.float32),
        pltpu.VMEM((_RW * _K + 16,), jnp.int32),
        pltpu.VMEM((_RW * _K + 16,), jnp.float32),
        pltpu.VMEM((_RW + 16,), jnp.int32),
    ],
)(_sc_body)


def _tc_block(ray_ref, ray2_ref, amu_ref, a_ref, amumu_ref,
              w_ref, idx_ref, vnum_ref, t_ref):
    rx = ray_ref[:, 0:1]
    ry = ray_ref[:, 1:2]
    rz = ray_ref[:, 2:3]
    rx2 = ray2_ref[:, 0:1]
    ry2 = ray2_ref[:, 1:2]
    rz2 = ray2_ref[:, 2:3]
    ax = amu_ref[0:1, :]
    ay = amu_ref[1:2, :]
    az = amu_ref[2:3, :]
    a = a_ref[...]
    amumu = amumu_ref[...]

    muAr = (rx * ax + ry * ay) + rz * az
    rAr = (rx2 * a + ry2 * a) + rz2 * a
    t = muAr / rAr
    quad = amumu - (muAr * muAr) / rAr
    act = jnp.exp(-0.5 * quad)
    valid = (act > _THR) & (t > 0.0)
    act_m = jnp.where(valid, act, 0.0)
    vnum = jnp.minimum(jnp.sum(valid.astype(jnp.int32), axis=1, keepdims=True), _K)

    lane = jax.lax.broadcasted_iota(jnp.int32, (_RB, _N), 1)

    vals_l, idx_l, ts_l = [], [], []
    for _ in range(_K):
        mx = jnp.max(act_m, axis=1, keepdims=True)
        cand = jnp.where(act_m == mx, lane, _N)
        am = jnp.min(cand, axis=1, keepdims=True)
        sel = lane == am
        tk = jnp.sum(jnp.where(sel, t, 0.0), axis=1, keepdims=True)
        vals_l.append(mx)
        idx_l.append(am)
        ts_l.append(tk)
        act_m = jnp.where(sel, -1.0, act_m)

    vals = jnp.concatenate(vals_l, axis=1)
    idxs = jnp.concatenate(idx_l, axis=1)
    ts = jnp.concatenate(ts_l, axis=1)

    key = jnp.where(vals > 0.0, ts, jnp.inf)
    lane_k = jax.lax.broadcasted_iota(jnp.int32, (_RB, _K), 1)
    act_s = jnp.zeros((_RB, _K), jnp.float32)
    idx_s = jnp.zeros((_RB, _K), jnp.int32)
    t_s = jnp.zeros((_RB, _K), jnp.float32)
    for i in range(_K):
        ki = key[:, i:i + 1]
        less = jnp.sum((key < ki).astype(jnp.int32), axis=1, keepdims=True)
        eqb = jnp.sum(((key == ki) & (lane_k < i)).astype(jnp.int32), axis=1, keepdims=True)
        rank = less + eqb
        oh = lane_k == rank
        act_s = jnp.where(oh, vals[:, i:i + 1], act_s)
        idx_s = jnp.where(oh, idxs[:, i:i + 1], idx_s)
        t_s = jnp.where(oh, ts[:, i:i + 1], t_s)

    alpha = jnp.clip(act_s, 0.0, 0.9999)
    trans = jnp.ones((_RB, 1), jnp.float32)
    w_cols = []
    for i in range(_K):
        al = alpha[:, i:i + 1]
        w_cols.append(al * trans)
        trans = trans * (1.0 - al)
    w = jnp.concatenate(w_cols, axis=1)

    w_ref[...] = w
    idx_ref[...] = idx_s
    vnum_ref[...] = vnum
    t_ref[...] = jnp.where(act_s > 0.0, t_s, 0.0)


@jax.jit
def kernel(verts, sigmas, radians):
    del radians
    r = _ray_dirs()
    r2 = r * r
    mu = verts[0]
    A = 2.0 * sigmas
    muA = jnp.einsum('nij,nj->ni', A, mu)
    muAmu = jnp.sum(muA * mu, axis=-1)
    a = A[:, 0, 0]

    bf16 = jnp.bfloat16
    rays6 = jnp.concatenate([r, r2], axis=1).T.astype(bf16)
    raysP = r.astype(bf16)
    rays2P = r2.astype(bf16)
    gauss4 = jnp.concatenate([muA.T, a[None, :]], axis=0).astype(bf16)

    rays6f, raysPf, rays2Pf, gauss4f = pl.pallas_call(
        _prep_body,
        out_shape=[
            jax.ShapeDtypeStruct((6, _P), jnp.float32),
            jax.ShapeDtypeStruct((_P, 3), jnp.float32),
            jax.ShapeDtypeStruct((_P, 3), jnp.float32),
            jax.ShapeDtypeStruct((4, _N), jnp.float32),
        ],
    )(rays6, raysP, rays2P, gauss4)

    amumu_row = muAmu[None, :]

    w_sc, idx_sc, vnum_sc, t_sc = _sc_render(
        rays6f.reshape(-1), gauss4f.reshape(-1), muAmu)

    grid = _S // _RB
    w_tc, idx_tc, vnum_tc, t_tc = pl.pallas_call(
        _tc_block,
        grid=(grid,),
        in_specs=[
            pl.BlockSpec((_RB, 3), lambda i: (i, 0)),
            pl.BlockSpec((_RB, 3), lambda i: (i, 0)),
            pl.BlockSpec((3, _N), lambda i: (0, 0)),
            pl.BlockSpec((1, _N), lambda i: (0, 0)),
            pl.BlockSpec((1, _N), lambda i: (0, 0)),
        ],
        out_specs=[
            pl.BlockSpec((_RB, _K), lambda i: (i, 0)),
            pl.BlockSpec((_RB, _K), lambda i: (i, 0)),
            pl.BlockSpec((_RB, 1), lambda i: (i, 0)),
            pl.BlockSpec((_RB, _K), lambda i: (i, 0)),
        ],
        out_shape=[
            jax.ShapeDtypeStruct((_S, _K), jnp.float32),
            jax.ShapeDtypeStruct((_S, _K), jnp.int32),
            jax.ShapeDtypeStruct((_S, 1), jnp.int32),
            jax.ShapeDtypeStruct((_S, _K), jnp.float32),
        ],
    )(raysPf, rays2Pf, gauss4f[0:3, :], gauss4f[3:4, :], amumu_row)

    w = jnp.concatenate([w_tc, w_sc.reshape(_PSC, _K)], axis=0)
    idx = jnp.concatenate([idx_tc, idx_sc.reshape(_PSC, _K)], axis=0)
    vnum = jnp.concatenate([vnum_tc.reshape(_S), vnum_sc], axis=0)
    ts = jnp.concatenate([t_tc, t_sc.reshape(_PSC, _K)], axis=0)

    return (w.reshape(1, _H, _W, _K),
            idx.reshape(1, _H, _W, _K),
            vnum.reshape(1, _H, _W),
            ts.reshape(1, _H, _W, _K))


# speedup vs baseline: 1.0059x; 1.0059x over previous
import functools

import jax
import jax.numpy as jnp
import numpy as np
from jax import lax
from jax.experimental import pallas as pl
from jax.experimental.pallas import tpu as pltpu
from jax.experimental.pallas import tpu_sc as plsc

_H = 128
_W = 128
_FOCAL = 150.0
_CX = 63.5
_CY = 63.5
_K = 20
_THR = 0.01
_N = 1024
_P = _H * _W
_S = 9728
_PSC = _P - _S
_NW = 32
_RW = _PSC // _NW
_NC = _N // 16
_CAP = _N + 16
_RB = 512
_F0 = np.float32(0.0)
_F1 = np.float32(1.0)


def _ray_dirs():
    ys, xs = jnp.meshgrid(jnp.arange(_H, dtype=jnp.float32),
                          jnp.arange(_W, dtype=jnp.float32), indexing='ij')
    d = jnp.stack([(xs - _CX) / _FOCAL, (ys - _CY) / _FOCAL, jnp.ones_like(xs)], axis=-1)
    d = d / jnp.linalg.norm(d, axis=-1, keepdims=True)
    return d.reshape(-1, 3)


def _prep_body(r_ref, rp_ref, r2p_ref, g_ref,
               ro_ref, rpo_ref, r2po_ref, go_ref):
    ro_ref[...] = r_ref[...].astype(jnp.float32)
    rpo_ref[...] = rp_ref[...].astype(jnp.float32)
    r2po_ref[...] = r2p_ref[...].astype(jnp.float32)
    go_ref[...] = g_ref[...].astype(jnp.float32)


def _ld1(ref, off):
    return ref[pl.ds(off, 16)][0]


def _sc_body(rays_hbm, gauss_hbm, amm_hbm,
             w_hbm, idx_hbm, vnum_hbm, t_hbm,
             rxv, ryv, rzv, rx2v, ry2v, rz2v,
             gxv, gyv, gzv, gav, ammv,
             rowA, rowT, keyb, selAb, selTb, selIb, sortAb, sortTb,
             wst, ist, tst, vnst):
    wid = lax.axis_index("s") * 2 + lax.axis_index("c")
    base = wid * _RW

    for j, ref in enumerate((rxv, ryv, rzv, rx2v, ry2v, rz2v)):
        pltpu.sync_copy(rays_hbm.at[pl.ds(j * _P + _S + base, _RW)],
                        ref.at[pl.ds(0, _RW)])
    for j, ref in enumerate((gxv, gyv, gzv, gav)):
        pltpu.sync_copy(gauss_hbm.at[pl.ds(j * _N, _N)], ref)
    pltpu.sync_copy(amm_hbm, ammv)

    iota = lax.iota(jnp.int32, 16)
    iota_hi = iota + 16
    zf = jnp.zeros((16,), jnp.float32)
    zi = jnp.zeros((16,), jnp.int32)
    one_i = jnp.full((16,), 1, jnp.int32)
    perms = [iota ^ sh for sh in (8, 4, 2, 1)]
    _pib = lax.GatherScatterMode.PROMISE_IN_BOUNDS

    def _shuf(v, perm):
        return v.at[perm].get(mode=_pib)

    def _bfmax(v):
        for perm in perms:
            v = jnp.maximum(v, _shuf(v, perm))
        return v

    def _bfmin(v):
        for perm in perms:
            v = jnp.minimum(v, _shuf(v, perm))
        return v

    def _bfsum(v):
        for perm in perms:
            v = v + _shuf(v, perm)
        return v

    big_i = jnp.full((16,), 1 << 30, jnp.int32)
    big16 = jnp.full((16,), 16, jnp.int32)

    def ray_body(p, vn):
        rx = rxv[pl.ds(p, 16)][0]
        ry = ryv[pl.ds(p, 16)][0]
        rz = rzv[pl.ds(p, 16)][0]
        rx2 = rx2v[pl.ds(p, 16)][0]
        ry2 = ry2v[pl.ds(p, 16)][0]
        rz2 = rz2v[pl.ds(p, 16)][0]

        def chunk_body(c, carry):
            m0, m1, m2, m3 = carry
            b = c * 16
            vx = gxv[pl.ds(b, 16)]
            vy = gyv[pl.ds(b, 16)]
            vz = gzv[pl.ds(b, 16)]
            va = gav[pl.ds(b, 16)]
            vm = ammv[pl.ds(b, 16)]
            muAr = (rx * vx + ry * vy) + rz * vz
            rAr = (rx2 * va + ry2 * va) + rz2 * va
            tv = muAr / rAr
            quad = vm - (muAr * muAr) / rAr
            actv = jnp.exp(-0.5 * quad)
            ok = (actv > _THR) & (tv > 0.0)
            am = jnp.where(ok, actv, _F0)
            rowA[pl.ds(b, 16)] = am
            rowT[pl.ds(b, 16)] = tv
            cm = _bfmax(am)
            m0 = jnp.where(iota == c, cm, m0)
            m1 = jnp.where(iota == c - 16, cm, m1)
            m2 = jnp.where(iota == c - 32, cm, m2)
            m3 = jnp.where(iota == c - 48, cm, m3)
            return (m0, m1, m2, m3)

        m0, m1, m2, m3 = lax.fori_loop(
            0, _NC, chunk_body, (zf, zf, zf, zf))

        def sel_body(k, carry):
            m0, m1, m2, m3, sA0, sA1, sT0, sT1, sI0, sI1 = carry
            mx = _bfmax(jnp.maximum(jnp.maximum(m0, m1), jnp.maximum(m2, m3)))
            c0 = jnp.where(m0 == mx, iota, big_i)
            c1 = jnp.where(m1 == mx, iota + 16, big_i)
            c2 = jnp.where(m2 == mx, iota + 32, big_i)
            c3 = jnp.where(m3 == mx, iota + 48, big_i)
            cs_vec = _bfmin(jnp.minimum(jnp.minimum(c0, c1), jnp.minimum(c2, c3)))
            cs = cs_vec[0]
            bb = cs * 16
            v = rowA[pl.ds(bb, 16)]
            ln_vec = _bfmin(jnp.where(v == mx, iota, big16))
            pos_vec = cs_vec * 16 + ln_vec
            tld = rowT[pl.ds(bb, 16)]
            tsel = _shuf(tld, ln_vec)
            v2 = jnp.where(iota == ln_vec, np.float32(-1.0), v)
            rowA[pl.ds(bb, 16)] = v2
            nm = _bfmax(v2)
            m0 = jnp.where(iota == cs, nm, m0)
            m1 = jnp.where(iota == cs - 16, nm, m1)
            m2 = jnp.where(iota == cs - 32, nm, m2)
            m3 = jnp.where(iota == cs - 48, nm, m3)
            sA0 = jnp.where(iota == k, mx, sA0)
            sA1 = jnp.where(iota_hi == k, mx, sA1)
            sT0 = jnp.where(iota == k, tsel, sT0)
            sT1 = jnp.where(iota_hi == k, tsel, sT1)
            sI0 = jnp.where(iota == k, pos_vec, sI0)
            sI1 = jnp.where(iota_hi == k, pos_vec, sI1)
            return (m0, m1, m2, m3, sA0, sA1, sT0, sT1, sI0, sI1)

        (_, _, _, _, sA0, sA1, sT0, sT1, sI0, sI1) = lax.fori_loop(
            0, _K, sel_body, (m0, m1, m2, m3, zf, zf, zf, zf, zi, zi))

        nvalid = _bfsum(jnp.where(sA0 > 0.0, one_i, zi)
                        + jnp.where((sA1 > 0.0) & (iota_hi < _K), one_i, zi))

        inf = np.float32(np.inf)
        key0 = jnp.where(sA0 > 0.0, sT0, inf)
        key1 = jnp.where((sA1 > 0.0) & (iota_hi < _K), sT1, inf)
        keyb[pl.ds(0, 16)] = key0
        keyb[pl.ds(16, 16)] = key1
        selAb[pl.ds(0, 16)] = sA0
        selAb[pl.ds(16, 16)] = sA1
        selTb[pl.ds(0, 16)] = sT0
        selTb[pl.ds(16, 16)] = sT1
        selIb[pl.ds(0, 16)] = sI0
        selIb[pl.ds(16, 16)] = sI1

        def rank_body(i, carry):
            oA0, oA1, oT0, oT1, oI0, oI1 = carry
            ki = _ld1(keyb, i)
            cnt = (jnp.where(key0 < ki, one_i, zi)
                   + jnp.where(key1 < ki, one_i, zi)
                   + jnp.where((key0 == ki) & (iota < i), one_i, zi)
                   + jnp.where((key1 == ki) & (iota_hi < i), one_i, zi))
            rank_vec = _bfsum(cnt)
            ai = _ld1(selAb, i)
            ti = _ld1(selTb, i)
            ii = _ld1(selIb, i)
            oA0 = jnp.where(iota == rank_vec, ai, oA0)
            oA1 = jnp.where(iota_hi == rank_vec, ai, oA1)
            oT0 = jnp.where(iota == rank_vec, ti, oT0)
            oT1 = jnp.where(iota_hi == rank_vec, ti, oT1)
            oI0 = jnp.where(iota == rank_vec, ii, oI0)
            oI1 = jnp.where(iota_hi == rank_vec, ii, oI1)
            return (oA0, oA1, oT0, oT1, oI0, oI1)

        oA0, oA1, oT0, oT1, oI0, oI1 = lax.fori_loop(
            0, _K, rank_body, (zf, zf, zf, zf, zi, zi))
        sortAb[pl.ds(0, 16)] = oA0
        sortAb[pl.ds(16, 16)] = oA1
        sortTb[pl.ds(0, 16)] = oT0
        sortTb[pl.ds(16, 16)] = oT1

        def comp_body(k2, carry):
            trans, w0, w1, t0, t1 = carry
            av = _ld1(sortAb, k2)
            tvs = _ld1(sortTb, k2)
            alpha = jnp.minimum(jnp.maximum(av, _F0), np.float32(0.9999))
            wv = alpha * trans
            tv2 = jnp.where(av > 0.0, tvs, _F0)
            w0 = jnp.where(iota == k2, wv, w0)
            w1 = jnp.where(iota_hi == k2, wv, w1)
            t0 = jnp.where(iota == k2, tv2, t0)
            t1 = jnp.where(iota_hi == k2, tv2, t1)
            return (trans * (1.0 - alpha), w0, w1, t0, t1)

        _, w0, w1, t0, t1 = lax.fori_loop(
            0, _K, comp_body, (_F1, zf, zf, zf, zf))

        obase = p * _K
        wst[pl.ds(obase, 16)] = w0
        wst[pl.ds(obase + 16, 16)] = w1
        ist[pl.ds(obase, 16)] = oI0
        ist[pl.ds(obase + 16, 16)] = oI1
        tst[pl.ds(obase, 16)] = t0
        tst[pl.ds(obase + 16, 16)] = t1

        vn = jnp.where(iota == (p & 15), nvalid, vn)
        vnst[pl.ds((p >> 4) * 16, 16)] = vn
        return vn

    lax.fori_loop(0, _RW, ray_body, zi)

    pltpu.sync_copy(wst.at[pl.ds(0, _RW * _K)], w_hbm.at[pl.ds(base * _K, _RW * _K)])
    pltpu.sync_copy(ist.at[pl.ds(0, _RW * _K)], idx_hbm.at[pl.ds(base * _K, _RW * _K)])
    pltpu.sync_copy(tst.at[pl.ds(0, _RW * _K)], t_hbm.at[pl.ds(base * _K, _RW * _K)])
    pltpu.sync_copy(vnst.at[pl.ds(0, _RW)], vnum_hbm.at[pl.ds(base, _RW)])


_sc_render = functools.partial(
    pl.kernel,
    mesh=plsc.VectorSubcoreMesh(core_axis_name="c", subcore_axis_name="s"),
    out_type=[
        jax.ShapeDtypeStruct((_PSC * _K,), jnp.float32),
        jax.ShapeDtypeStruct((_PSC * _K,), jnp.int32),
        jax.ShapeDtypeStruct((_PSC,), jnp.int32),
        jax.ShapeDtypeStruct((_PSC * _K,), jnp.float32),
    ],
    scratch_types=[
        pltpu.VMEM((_RW + 16,), jnp.float32),
        pltpu.VMEM((_RW + 16,), jnp.float32),
        pltpu.VMEM((_RW + 16,), jnp.float32),
        pltpu.VMEM((_RW + 16,), jnp.float32),
        pltpu.VMEM((_RW + 16,), jnp.float32),
        pltpu.VMEM((_RW + 16,), jnp.float32),
        pltpu.VMEM((_N,), jnp.float32),
        pltpu.VMEM((_N,), jnp.float32),
        pltpu.VMEM((_N,), jnp.float32),
        pltpu.VMEM((_N,), jnp.float32),
        pltpu.VMEM((_N,), jnp.float32),
        pltpu.VMEM((_CAP,), jnp.float32),
        pltpu.VMEM((_CAP,), jnp.float32),
        pltpu.VMEM((48,), jnp.float32),
        pltpu.VMEM((48,), jnp.float32),
        pltpu.VMEM((48,), jnp.float32),
        pltpu.VMEM((48,), jnp.int32),
        pltpu.VMEM((48,), jnp.float32),
        pltpu.VMEM((48,), jnp.float32),
        pltpu.VMEM((_RW * _K + 16,), jnp.float32),
        pltpu.VMEM((_RW * _K + 16,), jnp.int32),
        pltpu.VMEM((_RW * _K + 16,), jnp.float32),
        pltpu.VMEM((_RW + 16,), jnp.int32),
    ],
)(_sc_body)


def _tc_block(ray_ref, ray2_ref, amu_ref, a_ref, amumu_ref,
              w_ref, idx_ref, vnum_ref, t_ref):
    rx = ray_ref[:, 0:1]
    ry = ray_ref[:, 1:2]
    rz = ray_ref[:, 2:3]
    rx2 = ray2_ref[:, 0:1]
    ry2 = ray2_ref[:, 1:2]
    rz2 = ray2_ref[:, 2:3]
    ax = amu_ref[0:1, :]
    ay = amu_ref[1:2, :]
    az = amu_ref[2:3, :]
    a = a_ref[...]
    amumu = amumu_ref[...]

    muAr = (rx * ax + ry * ay) + rz * az
    rAr = (rx2 * a + ry2 * a) + rz2 * a
    t = muAr / rAr
    quad = amumu - (muAr * muAr) / rAr
    act = jnp.exp(-0.5 * quad)
    valid = (act > _THR) & (t > 0.0)
    act_m = jnp.where(valid, act, 0.0)
    vnum = jnp.minimum(jnp.sum(valid.astype(jnp.int32), axis=1, keepdims=True), _K)

    lane = jax.lax.broadcasted_iota(jnp.int32, (_RB, _N), 1)

    vals_l, idx_l, ts_l = [], [], []
    for _ in range(_K):
        mx = jnp.max(act_m, axis=1, keepdims=True)
        cand = jnp.where(act_m == mx, lane, _N)
        am = jnp.min(cand, axis=1, keepdims=True)
        sel = lane == am
        tk = jnp.sum(jnp.where(sel, t, 0.0), axis=1, keepdims=True)
        vals_l.append(mx)
        idx_l.append(am)
        ts_l.append(tk)
        act_m = jnp.where(sel, -1.0, act_m)

    vals = jnp.concatenate(vals_l, axis=1)
    idxs = jnp.concatenate(idx_l, axis=1)
    ts = jnp.concatenate(ts_l, axis=1)

    key = jnp.where(vals > 0.0, ts, jnp.inf)
    lane_k = jax.lax.broadcasted_iota(jnp.int32, (_RB, _K), 1)
    act_s = jnp.zeros((_RB, _K), jnp.float32)
    idx_s = jnp.zeros((_RB, _K), jnp.int32)
    t_s = jnp.zeros((_RB, _K), jnp.float32)
    for i in range(_K):
        ki = key[:, i:i + 1]
        less = jnp.sum((key < ki).astype(jnp.int32), axis=1, keepdims=True)
        eqb = jnp.sum(((key == ki) & (lane_k < i)).astype(jnp.int32), axis=1, keepdims=True)
        rank = less + eqb
        oh = lane_k == rank
        act_s = jnp.where(oh, vals[:, i:i + 1], act_s)
        idx_s = jnp.where(oh, idxs[:, i:i + 1], idx_s)
        t_s = jnp.where(oh, ts[:, i:i + 1], t_s)

    alpha = jnp.clip(act_s, 0.0, 0.9999)
    trans = jnp.ones((_RB, 1), jnp.float32)
    w_cols = []
    for i in range(_K):
        al = alpha[:, i:i + 1]
        w_cols.append(al * trans)
        trans = trans * (1.0 - al)
    w = jnp.concatenate(w_cols, axis=1)

    w_ref[...] = w
    idx_ref[...] = idx_s
    vnum_ref[...] = vnum
    t_ref[...] = jnp.where(act_s > 0.0, t_s, 0.0)


@jax.jit
def kernel(verts, sigmas, radians):
    del radians
    r = _ray_dirs()
    r2 = r * r
    mu = verts[0]
    A = 2.0 * sigmas
    muA = jnp.einsum('nij,nj->ni', A, mu)
    muAmu = jnp.sum(muA * mu, axis=-1)
    a = A[:, 0, 0]

    bf16 = jnp.bfloat16
    rays6 = jnp.concatenate([r, r2], axis=1).T.astype(bf16)
    raysP = r.astype(bf16)
    rays2P = r2.astype(bf16)
    gauss4 = jnp.concatenate([muA.T, a[None, :]], axis=0).astype(bf16)

    rays6f, raysPf, rays2Pf, gauss4f = pl.pallas_call(
        _prep_body,
        out_shape=[
            jax.ShapeDtypeStruct((6, _P), jnp.float32),
            jax.ShapeDtypeStruct((_P, 3), jnp.float32),
            jax.ShapeDtypeStruct((_P, 3), jnp.float32),
            jax.ShapeDtypeStruct((4, _N), jnp.float32),
        ],
    )(rays6, raysP, rays2P, gauss4)

    amumu_row = muAmu[None, :]

    w_sc, idx_sc, vnum_sc, t_sc = _sc_render(
        rays6f.reshape(-1), gauss4f.reshape(-1), muAmu)

    grid = _S // _RB
    w_tc, idx_tc, vnum_tc, t_tc = pl.pallas_call(
        _tc_block,
        grid=(grid,),
        in_specs=[
            pl.BlockSpec((_RB, 3), lambda i: (i, 0)),
            pl.BlockSpec((_RB, 3), lambda i: (i, 0)),
            pl.BlockSpec((3, _N), lambda i: (0, 0)),
            pl.BlockSpec((1, _N), lambda i: (0, 0)),
            pl.BlockSpec((1, _N), lambda i: (0, 0)),
        ],
        out_specs=[
            pl.BlockSpec((_RB, _K), lambda i: (i, 0)),
            pl.BlockSpec((_RB, _K), lambda i: (i, 0)),
            pl.BlockSpec((_RB, 1), lambda i: (i, 0)),
            pl.BlockSpec((_RB, _K), lambda i: (i, 0)),
        ],
        out_shape=[
            jax.ShapeDtypeStruct((_S, _K), jnp.float32),
            jax.ShapeDtypeStruct((_S, _K), jnp.int32),
            jax.ShapeDtypeStruct((_S, 1), jnp.int32),
            jax.ShapeDtypeStruct((_S, _K), jnp.float32),
        ],
    )(raysPf, rays2Pf, gauss4f[0:3, :], gauss4f[3:4, :], amumu_row)

    w = jnp.concatenate([w_tc, w_sc.reshape(_PSC, _K)], axis=0)
    idx = jnp.concatenate([idx_tc, idx_sc.reshape(_PSC, _K)], axis=0)
    vnum = jnp.concatenate([vnum_tc.reshape(_S), vnum_sc], axis=0)
    ts = jnp.concatenate([t_tc, t_sc.reshape(_PSC, _K)], axis=0)

    return (w.reshape(1, _H, _W, _K),
            idx.reshape(1, _H, _W, _K),
            vnum.reshape(1, _H, _W),
            ts.reshape(1, _H, _W, _K))


# speedup vs baseline: 1.0519x; 1.0456x over previous
import functools

import jax
import jax.numpy as jnp
import numpy as np
from jax import lax
from jax.experimental import pallas as pl
from jax.experimental.pallas import tpu as pltpu
from jax.experimental.pallas import tpu_sc as plsc

_H = 128
_W = 128
_FOCAL = 150.0
_CX = 63.5
_CY = 63.5
_K = 20
_THR = 0.01
_N = 1024
_P = _H * _W
_S = 9728
_PSC = _P - _S
_NW = 32
_RW = _PSC // _NW
_NC = _N // 16
_CAP = _N + 16
_RB = 512
_F0 = np.float32(0.0)
_F1 = np.float32(1.0)


def _ray_dirs():
    ys, xs = jnp.meshgrid(jnp.arange(_H, dtype=jnp.float32),
                          jnp.arange(_W, dtype=jnp.float32), indexing='ij')
    d = jnp.stack([(xs - _CX) / _FOCAL, (ys - _CY) / _FOCAL, jnp.ones_like(xs)], axis=-1)
    d = d / jnp.linalg.norm(d, axis=-1, keepdims=True)
    return d.reshape(-1, 3)


def _prep_body(r_ref, g_ref, ro_ref, go_ref):
    ro_ref[...] = r_ref[...].astype(jnp.float32)
    go_ref[...] = g_ref[...].astype(jnp.float32)


def _ld1(ref, off):
    return ref[pl.ds(off, 16)][0]


def _sc_body(rays_hbm, gauss_hbm, amm_hbm,
             w_hbm, idx_hbm, vnum_hbm, t_hbm,
             rxv, ryv, rzv, rx2v, ry2v, rz2v,
             gxv, gyv, gzv, gav, ammv,
             rowA, rowT, keyb, selAb, selTb, selIb, sortAb, sortTb,
             wst, ist, tst, vnst):
    wid = lax.axis_index("s") * 2 + lax.axis_index("c")
    base = wid * _RW

    for j, ref in enumerate((rxv, ryv, rzv, rx2v, ry2v, rz2v)):
        pltpu.sync_copy(rays_hbm.at[pl.ds(j * _P + _S + base, _RW)],
                        ref.at[pl.ds(0, _RW)])
    for j, ref in enumerate((gxv, gyv, gzv, gav)):
        pltpu.sync_copy(gauss_hbm.at[pl.ds(j * _N, _N)], ref)
    pltpu.sync_copy(amm_hbm, ammv)

    iota = lax.iota(jnp.int32, 16)
    iota_hi = iota + 16
    zf = jnp.zeros((16,), jnp.float32)
    zi = jnp.zeros((16,), jnp.int32)
    one_i = jnp.full((16,), 1, jnp.int32)
    perms = [iota ^ sh for sh in (8, 4, 2, 1)]
    _pib = lax.GatherScatterMode.PROMISE_IN_BOUNDS

    def _shuf(v, perm):
        return v.at[perm].get(mode=_pib)

    def _bfmax(v):
        for perm in perms:
            v = jnp.maximum(v, _shuf(v, perm))
        return v

    def _bfmin(v):
        for perm in perms:
            v = jnp.minimum(v, _shuf(v, perm))
        return v

    def _bfsum(v):
        for perm in perms:
            v = v + _shuf(v, perm)
        return v

    big_i = jnp.full((16,), 1 << 30, jnp.int32)
    big16 = jnp.full((16,), 16, jnp.int32)

    def ray_body(p, vn):
        rx = rxv[pl.ds(p, 16)][0]
        ry = ryv[pl.ds(p, 16)][0]
        rz = rzv[pl.ds(p, 16)][0]
        rx2 = rx2v[pl.ds(p, 16)][0]
        ry2 = ry2v[pl.ds(p, 16)][0]
        rz2 = rz2v[pl.ds(p, 16)][0]

        def chunk_body(c, carry):
            m0, m1, m2, m3 = carry
            b = c * 16
            vx = gxv[pl.ds(b, 16)]
            vy = gyv[pl.ds(b, 16)]
            vz = gzv[pl.ds(b, 16)]
            va = gav[pl.ds(b, 16)]
            vm = ammv[pl.ds(b, 16)]
            muAr = (rx * vx + ry * vy) + rz * vz
            rAr = (rx2 * va + ry2 * va) + rz2 * va
            tv = muAr / rAr
            quad = vm - (muAr * muAr) / rAr
            actv = jnp.exp(-0.5 * quad)
            ok = (actv > _THR) & (tv > 0.0)
            am = jnp.where(ok, actv, _F0)
            rowA[pl.ds(b, 16)] = am
            rowT[pl.ds(b, 16)] = tv
            cm = _bfmax(am)
            m0 = jnp.where(iota == c, cm, m0)
            m1 = jnp.where(iota == c - 16, cm, m1)
            m2 = jnp.where(iota == c - 32, cm, m2)
            m3 = jnp.where(iota == c - 48, cm, m3)
            return (m0, m1, m2, m3)

        m0, m1, m2, m3 = lax.fori_loop(
            0, _NC, chunk_body, (zf, zf, zf, zf))

        def sel_body(k, carry):
            m0, m1, m2, m3, sA0, sA1, sT0, sT1, sI0, sI1 = carry
            mx = _bfmax(jnp.maximum(jnp.maximum(m0, m1), jnp.maximum(m2, m3)))
            c0 = jnp.where(m0 == mx, iota, big_i)
            c1 = jnp.where(m1 == mx, iota + 16, big_i)
            c2 = jnp.where(m2 == mx, iota + 32, big_i)
            c3 = jnp.where(m3 == mx, iota + 48, big_i)
            cs_vec = _bfmin(jnp.minimum(jnp.minimum(c0, c1), jnp.minimum(c2, c3)))
            cs = cs_vec[0]
            bb = cs * 16
            v = rowA[pl.ds(bb, 16)]
            ln_vec = _bfmin(jnp.where(v == mx, iota, big16))
            pos_vec = cs_vec * 16 + ln_vec
            tld = rowT[pl.ds(bb, 16)]
            tsel = _shuf(tld, ln_vec)
            v2 = jnp.where(iota == ln_vec, np.float32(-1.0), v)
            rowA[pl.ds(bb, 16)] = v2
            nm = _bfmax(v2)
            m0 = jnp.where(iota == cs, nm, m0)
            m1 = jnp.where(iota == cs - 16, nm, m1)
            m2 = jnp.where(iota == cs - 32, nm, m2)
            m3 = jnp.where(iota == cs - 48, nm, m3)
            sA0 = jnp.where(iota == k, mx, sA0)
            sA1 = jnp.where(iota_hi == k, mx, sA1)
            sT0 = jnp.where(iota == k, tsel, sT0)
            sT1 = jnp.where(iota_hi == k, tsel, sT1)
            sI0 = jnp.where(iota == k, pos_vec, sI0)
            sI1 = jnp.where(iota_hi == k, pos_vec, sI1)
            return (m0, m1, m2, m3, sA0, sA1, sT0, sT1, sI0, sI1)

        (_, _, _, _, sA0, sA1, sT0, sT1, sI0, sI1) = lax.fori_loop(
            0, _K, sel_body, (m0, m1, m2, m3, zf, zf, zf, zf, zi, zi))

        nvalid = _bfsum(jnp.where(sA0 > 0.0, one_i, zi)
                        + jnp.where((sA1 > 0.0) & (iota_hi < _K), one_i, zi))

        inf = np.float32(np.inf)
        key0 = jnp.where(sA0 > 0.0, sT0, inf)
        key1 = jnp.where((sA1 > 0.0) & (iota_hi < _K), sT1, inf)
        keyb[pl.ds(0, 16)] = key0
        keyb[pl.ds(16, 16)] = key1
        selAb[pl.ds(0, 16)] = sA0
        selAb[pl.ds(16, 16)] = sA1
        selTb[pl.ds(0, 16)] = sT0
        selTb[pl.ds(16, 16)] = sT1
        selIb[pl.ds(0, 16)] = sI0
        selIb[pl.ds(16, 16)] = sI1

        def rank_body(i, carry):
            oA0, oA1, oT0, oT1, oI0, oI1 = carry
            ki = _ld1(keyb, i)
            cnt = (jnp.where(key0 < ki, one_i, zi)
                   + jnp.where(key1 < ki, one_i, zi)
                   + jnp.where((key0 == ki) & (iota < i), one_i, zi)
                   + jnp.where((key1 == ki) & (iota_hi < i), one_i, zi))
            rank_vec = _bfsum(cnt)
            ai = _ld1(selAb, i)
            ti = _ld1(selTb, i)
            ii = _ld1(selIb, i)
            oA0 = jnp.where(iota == rank_vec, ai, oA0)
            oA1 = jnp.where(iota_hi == rank_vec, ai, oA1)
            oT0 = jnp.where(iota == rank_vec, ti, oT0)
            oT1 = jnp.where(iota_hi == rank_vec, ti, oT1)
            oI0 = jnp.where(iota == rank_vec, ii, oI0)
            oI1 = jnp.where(iota_hi == rank_vec, ii, oI1)
            return (oA0, oA1, oT0, oT1, oI0, oI1)

        oA0, oA1, oT0, oT1, oI0, oI1 = lax.fori_loop(
            0, _K, rank_body, (zf, zf, zf, zf, zi, zi))
        sortAb[pl.ds(0, 16)] = oA0
        sortAb[pl.ds(16, 16)] = oA1
        sortTb[pl.ds(0, 16)] = oT0
        sortTb[pl.ds(16, 16)] = oT1

        def comp_body(k2, carry):
            trans, w0, w1, t0, t1 = carry
            av = _ld1(sortAb, k2)
            tvs = _ld1(sortTb, k2)
            alpha = jnp.minimum(jnp.maximum(av, _F0), np.float32(0.9999))
            wv = alpha * trans
            tv2 = jnp.where(av > 0.0, tvs, _F0)
            w0 = jnp.where(iota == k2, wv, w0)
            w1 = jnp.where(iota_hi == k2, wv, w1)
            t0 = jnp.where(iota == k2, tv2, t0)
            t1 = jnp.where(iota_hi == k2, tv2, t1)
            return (trans * (1.0 - alpha), w0, w1, t0, t1)

        _, w0, w1, t0, t1 = lax.fori_loop(
            0, _K, comp_body, (_F1, zf, zf, zf, zf))

        obase = p * _K
        wst[pl.ds(obase, 16)] = w0
        wst[pl.ds(obase + 16, 16)] = w1
        ist[pl.ds(obase, 16)] = oI0
        ist[pl.ds(obase + 16, 16)] = oI1
        tst[pl.ds(obase, 16)] = t0
        tst[pl.ds(obase + 16, 16)] = t1

        vn = jnp.where(iota == (p & 15), nvalid, vn)
        vnst[pl.ds((p >> 4) * 16, 16)] = vn
        return vn

    lax.fori_loop(0, _RW, ray_body, zi)

    pltpu.sync_copy(wst.at[pl.ds(0, _RW * _K)], w_hbm.at[pl.ds(base * _K, _RW * _K)])
    pltpu.sync_copy(ist.at[pl.ds(0, _RW * _K)], idx_hbm.at[pl.ds(base * _K, _RW * _K)])
    pltpu.sync_copy(tst.at[pl.ds(0, _RW * _K)], t_hbm.at[pl.ds(base * _K, _RW * _K)])
    pltpu.sync_copy(vnst.at[pl.ds(0, _RW)], vnum_hbm.at[pl.ds(base, _RW)])


_sc_render = functools.partial(
    pl.kernel,
    mesh=plsc.VectorSubcoreMesh(core_axis_name="c", subcore_axis_name="s"),
    out_type=[
        jax.ShapeDtypeStruct((_PSC * _K,), jnp.float32),
        jax.ShapeDtypeStruct((_PSC * _K,), jnp.int32),
        jax.ShapeDtypeStruct((_PSC,), jnp.int32),
        jax.ShapeDtypeStruct((_PSC * _K,), jnp.float32),
    ],
    scratch_types=[
        pltpu.VMEM((_RW + 16,), jnp.float32),
        pltpu.VMEM((_RW + 16,), jnp.float32),
        pltpu.VMEM((_RW + 16,), jnp.float32),
        pltpu.VMEM((_RW + 16,), jnp.float32),
        pltpu.VMEM((_RW + 16,), jnp.float32),
        pltpu.VMEM((_RW + 16,), jnp.float32),
        pltpu.VMEM((_N,), jnp.float32),
        pltpu.VMEM((_N,), jnp.float32),
        pltpu.VMEM((_N,), jnp.float32),
        pltpu.VMEM((_N,), jnp.float32),
        pltpu.VMEM((_N,), jnp.float32),
        pltpu.VMEM((_CAP,), jnp.float32),
        pltpu.VMEM((_CAP,), jnp.float32),
        pltpu.VMEM((48,), jnp.float32),
        pltpu.VMEM((48,), jnp.float32),
        pltpu.VMEM((48,), jnp.float32),
        pltpu.VMEM((48,), jnp.int32),
        pltpu.VMEM((48,), jnp.float32),
        pltpu.VMEM((48,), jnp.float32),
        pltpu.VMEM((_RW * _K + 16,), jnp.float32),
        pltpu.VMEM((_RW * _K + 16,), jnp.int32),
        pltpu.VMEM((_RW * _K + 16,), jnp.float32),
        pltpu.VMEM((_RW + 16,), jnp.int32),
    ],
)(_sc_body)


def _tc_block(ray_ref, ray2_ref, amu_ref, a_ref, amumu_ref,
              w_ref, idx_ref, vnum_ref, t_ref):
    f = jnp.float32
    rx = ray_ref[:, 0:1].astype(f)
    ry = ray_ref[:, 1:2].astype(f)
    rz = ray_ref[:, 2:3].astype(f)
    rx2 = ray2_ref[:, 0:1].astype(f)
    ry2 = ray2_ref[:, 1:2].astype(f)
    rz2 = ray2_ref[:, 2:3].astype(f)
    ax = amu_ref[0:1, :].astype(f)
    ay = amu_ref[1:2, :].astype(f)
    az = amu_ref[2:3, :].astype(f)
    a = a_ref[...].astype(f)
    amumu = amumu_ref[...]

    muAr = (rx * ax + ry * ay) + rz * az
    rAr = (rx2 * a + ry2 * a) + rz2 * a
    t = muAr / rAr
    quad = amumu - (muAr * muAr) / rAr
    act = jnp.exp(-0.5 * quad)
    valid = (act > _THR) & (t > 0.0)
    act_m = jnp.where(valid, act, 0.0)
    vnum = jnp.minimum(jnp.sum(valid.astype(jnp.int32), axis=1, keepdims=True), _K)

    lane = jax.lax.broadcasted_iota(jnp.int32, (_RB, _N), 1)

    vals_l, idx_l, ts_l = [], [], []
    for _ in range(_K):
        mx = jnp.max(act_m, axis=1, keepdims=True)
        cand = jnp.where(act_m == mx, lane, _N)
        am = jnp.min(cand, axis=1, keepdims=True)
        sel = lane == am
        tk = jnp.sum(jnp.where(sel, t, 0.0), axis=1, keepdims=True)
        vals_l.append(mx)
        idx_l.append(am)
        ts_l.append(tk)
        act_m = jnp.where(sel, -1.0, act_m)

    vals = jnp.concatenate(vals_l, axis=1)
    idxs = jnp.concatenate(idx_l, axis=1)
    ts = jnp.concatenate(ts_l, axis=1)

    key = jnp.where(vals > 0.0, ts, jnp.inf)
    lane_k = jax.lax.broadcasted_iota(jnp.int32, (_RB, _K), 1)
    act_s = jnp.zeros((_RB, _K), jnp.float32)
    idx_s = jnp.zeros((_RB, _K), jnp.int32)
    t_s = jnp.zeros((_RB, _K), jnp.float32)
    for i in range(_K):
        ki = key[:, i:i + 1]
        less = jnp.sum((key < ki).astype(jnp.int32), axis=1, keepdims=True)
        eqb = jnp.sum(((key == ki) & (lane_k < i)).astype(jnp.int32), axis=1, keepdims=True)
        rank = less + eqb
        oh = lane_k == rank
        act_s = jnp.where(oh, vals[:, i:i + 1], act_s)
        idx_s = jnp.where(oh, idxs[:, i:i + 1], idx_s)
        t_s = jnp.where(oh, ts[:, i:i + 1], t_s)

    alpha = jnp.clip(act_s, 0.0, 0.9999)
    trans = jnp.ones((_RB, 1), jnp.float32)
    w_cols = []
    for i in range(_K):
        al = alpha[:, i:i + 1]
        w_cols.append(al * trans)
        trans = trans * (1.0 - al)
    w = jnp.concatenate(w_cols, axis=1)

    w_ref[...] = w
    idx_ref[...] = idx_s
    vnum_ref[...] = vnum
    t_ref[...] = jnp.where(act_s > 0.0, t_s, 0.0)


@jax.jit
def kernel(verts, sigmas, radians):
    del radians
    r = _ray_dirs()
    r2 = r * r
    mu = verts[0]
    A = 2.0 * sigmas
    muA = jnp.einsum('nij,nj->ni', A, mu)
    muAmu = jnp.sum(muA * mu, axis=-1)
    a = A[:, 0, 0]

    bf16 = jnp.bfloat16
    rays6 = jnp.concatenate([r, r2], axis=1).T.astype(bf16)
    raysP = r.astype(bf16)
    rays2P = r2.astype(bf16)
    gauss4 = jnp.concatenate([muA.T, a[None, :]], axis=0).astype(bf16)

    rays6f, gauss4f = pl.pallas_call(
        _prep_body,
        out_shape=[
            jax.ShapeDtypeStruct((6, _P), jnp.float32),
            jax.ShapeDtypeStruct((4, _N), jnp.float32),
        ],
    )(rays6, gauss4)
    amub16 = muA.astype(bf16).T
    ab16 = a.astype(bf16)[None, :]

    amumu_row = muAmu[None, :]

    w_sc, idx_sc, vnum_sc, t_sc = _sc_render(
        rays6f.reshape(-1), gauss4f.reshape(-1), muAmu)

    grid = _S // _RB
    w_tc, idx_tc, vnum_tc, t_tc = pl.pallas_call(
        _tc_block,
        grid=(grid,),
        in_specs=[
            pl.BlockSpec((_RB, 3), lambda i: (i, 0)),
            pl.BlockSpec((_RB, 3), lambda i: (i, 0)),
            pl.BlockSpec((3, _N), lambda i: (0, 0)),
            pl.BlockSpec((1, _N), lambda i: (0, 0)),
            pl.BlockSpec((1, _N), lambda i: (0, 0)),
        ],
        out_specs=[
            pl.BlockSpec((_RB, _K), lambda i: (i, 0)),
            pl.BlockSpec((_RB, _K), lambda i: (i, 0)),
            pl.BlockSpec((_RB, 1), lambda i: (i, 0)),
            pl.BlockSpec((_RB, _K), lambda i: (i, 0)),
        ],
        out_shape=[
            jax.ShapeDtypeStruct((_S, _K), jnp.float32),
            jax.ShapeDtypeStruct((_S, _K), jnp.int32),
            jax.ShapeDtypeStruct((_S, 1), jnp.int32),
            jax.ShapeDtypeStruct((_S, _K), jnp.float32),
        ],
    )(raysP, rays2P, amub16, ab16, amumu_row)

    w = jnp.concatenate([w_tc, w_sc.reshape(_PSC, _K)], axis=0)
    idx = jnp.concatenate([idx_tc, idx_sc.reshape(_PSC, _K)], axis=0)
    vnum = jnp.concatenate([vnum_tc.reshape(_S), vnum_sc], axis=0)
    ts = jnp.concatenate([t_tc, t_sc.reshape(_PSC, _K)], axis=0)

    return (w.reshape(1, _H, _W, _K),
            idx.reshape(1, _H, _W, _K),
            vnum.reshape(1, _H, _W),
            ts.reshape(1, _H, _W, _K))
